# baseline (speedup 1.0000x reference)
import jax
import jax.numpy as jnp
from jax.experimental import pallas as pl
from jax.experimental.pallas import tpu as pltpu
from jax.sharding import Mesh, PartitionSpec as P

_TILE_B = 2048
_NSPLIT = 2


def _mlp_kernel(*refs):
    x_refs = refs[:_NSPLIT]
    w1_ref, b1_ref, w2_ref, b2_ref, w3_ref, b3_ref, o_ref = refs[_NSPLIT:]
    w1 = w1_ref[...]
    for j, x_ref in enumerate(x_refs):
        xb = x_ref[...].astype(jnp.bfloat16)
        h = jnp.dot(xb, w1, preferred_element_type=jnp.float32)
        h = jnp.maximum(h + b1_ref[...], 0.0)
        h = jnp.dot(h, w2_ref[...], preferred_element_type=jnp.float32)
        h = jnp.maximum(h + b2_ref[...], 0.0)
        z = jnp.dot(h, w3_ref[...], preferred_element_type=jnp.float32)
        tb = x_ref.shape[0]
        o_ref[j * tb:(j + 1) * tb, :] = jax.nn.sigmoid(z + b3_ref[...])


def _forward_one_core(x, w1_t, b1_r, w2_t, b2_r, w3_t, b3_r):
    B, F = x.shape
    tile_b = min(_TILE_B, -(-B // 8) * 8)
    step_b = _NSPLIT * tile_b
    Bp = -(-B // step_b) * step_b
    if Bp != B:
        x = jnp.pad(x, ((0, Bp - B), (0, 0)))

    const = lambda i: (0, 0)
    flops = 2 * Bp * (F * 32 + 32 * 16 + 16)
    bytes_accessed = 4 * Bp * (F + 1) + 4 * (
        w1_t.size + b1_r.size + w2_t.size + b2_r.size + w3_t.size + b3_r.size)

    def x_map(j):
        return lambda i: (_NSPLIT * i + j, 0)

    out = pl.pallas_call(
        _mlp_kernel,
        out_shape=jax.ShapeDtypeStruct((Bp, 1), jnp.float32),
        grid=(Bp // step_b,),
        in_specs=[pl.BlockSpec((tile_b, F), x_map(j)) for j in range(_NSPLIT)]
        + [
            pl.BlockSpec(w1_t.shape, const),
            pl.BlockSpec(b1_r.shape, const),
            pl.BlockSpec(w2_t.shape, const),
            pl.BlockSpec(b2_r.shape, const),
            pl.BlockSpec(w3_t.shape, const),
            pl.BlockSpec(b3_r.shape, const),
        ],
        out_specs=pl.BlockSpec((step_b, 1), lambda i: (i, 0)),
        compiler_params=pltpu.CompilerParams(
            dimension_semantics=("arbitrary",),
        ),
        cost_estimate=pl.CostEstimate(
            flops=flops, transcendentals=Bp, bytes_accessed=bytes_accessed),
    )(*([x] * _NSPLIT), w1_t, b1_r, w2_t, b2_r, w3_t, b3_r)

    return out[:B] if Bp != B else out


def kernel(x, w1, b1, w2, b2, w3, b3):
    B, F = x.shape
    x = x.astype(jnp.float32)

    w1_t = w1.T.astype(jnp.bfloat16)
    w2_t, w3_t = w2.T, w3.T
    b1_r, b2_r, b3_r = (b.reshape(1, -1) for b in (b1, b2, b3))

    devs = [d for d in jax.devices() if d.platform == "tpu"]
    n_shards = 2 if (len(devs) >= 2 and B % 2 == 0) else 1
    if n_shards == 1:
        return _forward_one_core(x, w1_t, b1_r, w2_t, b2_r, w3_t, b3_r)

    mesh = Mesh(devs[:n_shards], ("c",))
    fwd = jax.shard_map(
        _forward_one_core,
        mesh=mesh,
        in_specs=(P("c", None),) + (P(None, None),) * 6,
        out_specs=P("c", None),
        check_vma=False,
    )
    return fwd(x, w1_t, b1_r, w2_t, b2_r, w3_t, b3_r)


# speedup vs baseline: 10.4574x; 10.4574x over previous
import jax
import jax.numpy as jnp
from jax.experimental import pallas as pl
from jax.experimental.pallas import tpu as pltpu

_TILE_B = 8192


def _mlp_kernel(x_ref, w1_ref, b1_ref, w2_ref, b2_ref, w3_ref, b3_ref, o_ref):
    xb = x_ref[...].astype(jnp.bfloat16)
    h = jnp.dot(xb, w1_ref[...], preferred_element_type=jnp.float32)
    h = jnp.maximum(h + b1_ref[...], 0.0)
    h = jnp.dot(h, w2_ref[...], preferred_element_type=jnp.float32)
    h = jnp.maximum(h + b2_ref[...], 0.0)
    z = jnp.dot(h, w3_ref[...], preferred_element_type=jnp.float32)
    o_ref[...] = jax.nn.sigmoid(z + b3_ref[...])


def kernel(x, w1, b1, w2, b2, w3, b3):
    B, F = x.shape
    x = x.astype(jnp.float32)

    tile_b = min(_TILE_B, -(-B // 8) * 8)
    Bp = -(-B // tile_b) * tile_b
    if Bp != B:
        x = jnp.pad(x, ((0, Bp - B), (0, 0)))

    w1_t = w1.T.astype(jnp.bfloat16)
    w2_t, w3_t = w2.T, w3.T
    b1_r, b2_r, b3_r = (b.reshape(1, -1) for b in (b1, b2, b3))

    const = lambda i: (0, 0)
    flops = 2 * Bp * (F * 32 + 32 * 16 + 16)
    bytes_accessed = 4 * Bp * (F + 1) + 4 * sum(
        a.size for a in (w1, b1, w2, b2, w3, b3))

    out = pl.pallas_call(
        _mlp_kernel,
        out_shape=jax.ShapeDtypeStruct((Bp, 1), jnp.float32),
        grid=(Bp // tile_b,),
        in_specs=[
            pl.BlockSpec((tile_b, F), lambda i: (i, 0)),
            pl.BlockSpec(w1_t.shape, const),
            pl.BlockSpec(b1_r.shape, const),
            pl.BlockSpec(w2_t.shape, const),
            pl.BlockSpec(b2_r.shape, const),
            pl.BlockSpec(w3_t.shape, const),
            pl.BlockSpec(b3_r.shape, const),
        ],
        out_specs=pl.BlockSpec((tile_b, 1), lambda i: (i, 0)),
        compiler_params=pltpu.CompilerParams(
            dimension_semantics=("arbitrary",),
        ),
        cost_estimate=pl.CostEstimate(
            flops=flops, transcendentals=Bp, bytes_accessed=bytes_accessed),
    )(x, w1_t, b1_r, w2_t, b2_r, w3_t, b3_r)

    return out[:B] if Bp != B else out
